# Initial kernel scaffold; baseline (speedup 1.0000x reference)
import functools

import jax
import jax.numpy as jnp
from jax import lax
from jax.experimental import pallas as pl
from jax.experimental.pallas import tpu as pltpu
from jax.experimental.pallas import tpu_sc as plsc

N = 10000
E = 320000
D = 128
R = 4

NC = 2
NS = 16
L = 16
NW = NC * NS
CH = 128
NCHUNK = E // CH
NJ = -(-NCHUNK // NW)
RPT = N // NS
ZR = 125


def _relmm(x, rel_weight):
    BN = 1000

    def body(x_ref, w_ref, o_ref):
        o_ref[...] = jnp.dot(x_ref[...], w_ref[0],
                             preferred_element_type=jnp.float32)

    return pl.pallas_call(
        body,
        grid=(R, N // BN),
        in_specs=[
            pl.BlockSpec((BN, D), lambda r, i: (i, 0)),
            pl.BlockSpec((1, D, D), lambda r, i: (r, 0, 0)),
        ],
        out_specs=pl.BlockSpec((BN, D), lambda r, i: (r * (N // BN) + i, 0)),
        out_shape=jax.ShapeDtypeStruct((R * N, D), jnp.float32),
    )(x, rel_weight)


def _sc_agg(xw, src, dst, et):
    mesh = plsc.VectorSubcoreMesh(core_axis_name="c", subcore_axis_name="s")

    @functools.partial(
        pl.kernel,
        mesh=mesh,
        out_type=jax.ShapeDtypeStruct((NC, N, D), jnp.float32),
        scratch_types=[
            pltpu.VMEM((CH,), jnp.int32),
            pltpu.VMEM((CH,), jnp.int32),
            pltpu.VMEM((CH,), jnp.int32),
            pltpu.VMEM((CH,), jnp.int32),
            pltpu.VMEM((CH, D), jnp.float32),
            pltpu.VMEM((ZR, D), jnp.float32),
            pltpu.VMEM_SHARED((N, D), jnp.float32),
            pltpu.SemaphoreType.DMA,
        ],
    )
    def k(xw_hbm, src_hbm, dst_hbm, et_hbm, out_hbm,
          src_v, dst_v, et_v, gidx_v, rows_v, stage_v, agg_sh, sem):
        c = lax.axis_index("c")
        s = lax.axis_index("s")
        wid = s * NC + c

        z16 = jnp.zeros((L,), jnp.float32)

        def zrow(i, carry):
            for j in range(D // L):
                stage_v[i, pl.ds(j * L, L)] = z16
            return carry

        lax.fori_loop(0, ZR, zrow, 0)
        for t in range(RPT // ZR):
            pltpu.sync_copy(stage_v, agg_sh.at[pl.ds(s * RPT + t * ZR, ZR)])
        plsc.subcore_barrier()

        def chunk(j, carry):
            cid = wid + j * NW

            @pl.when(cid < NCHUNK)
            def _():
                base = cid * CH
                pltpu.sync_copy(src_hbm.at[pl.ds(base, CH)], src_v)
                pltpu.sync_copy(et_hbm.at[pl.ds(base, CH)], et_v)
                pltpu.sync_copy(dst_hbm.at[pl.ds(base, CH)], dst_v)
                for i in range(CH // L):
                    sl = pl.ds(i * L, L)
                    gidx_v[sl] = et_v[sl] * N + src_v[sl]
                pltpu.async_copy(xw_hbm.at[gidx_v], rows_v, sem).wait()
                pltpu.sync_copy(rows_v, agg_sh.at[dst_v], add=True)

            return carry

        lax.fori_loop(0, NJ, chunk, 0)
        plsc.subcore_barrier()

        for t in range(RPT // ZR):
            r0 = s * RPT + t * ZR
            pltpu.sync_copy(agg_sh.at[pl.ds(r0, ZR)], stage_v)
            pltpu.sync_copy(stage_v, out_hbm.at[c, pl.ds(r0, ZR)])

    return k(xw, src, dst, et)


def _mlp(x, partials, lw, rb, w1xt, w1mt, b1, w2xt, w2mt, b2):
    BN = 1000

    def body(x_ref, p_ref, lw_ref, rb_ref, w1x_ref, w1m_ref, b1_ref,
             w2x_ref, w2m_ref, b2_ref, o_ref):
        xb = x_ref[...]
        msg = (p_ref[0] + p_ref[1]
               + jnp.dot(xb, lw_ref[...], preferred_element_type=jnp.float32)
               + rb_ref[...])
        h = (jnp.dot(xb, w1x_ref[...], preferred_element_type=jnp.float32)
             + jnp.dot(msg, w1m_ref[...], preferred_element_type=jnp.float32)
             + b1_ref[...])
        mid = jnp.tanh(h)
        o_ref[...] = (jnp.dot(xb, w2x_ref[...], preferred_element_type=jnp.float32)
                      + jnp.dot(mid, w2m_ref[...], preferred_element_type=jnp.float32)
                      + b2_ref[...])

    return pl.pallas_call(
        body,
        grid=(N // BN,),
        in_specs=[
            pl.BlockSpec((BN, D), lambda i: (i, 0)),
            pl.BlockSpec((NC, BN, D), lambda i: (0, i, 0)),
            pl.BlockSpec((D, D), lambda i: (0, 0)),
            pl.BlockSpec((1, D), lambda i: (0, 0)),
            pl.BlockSpec((D, 2 * D), lambda i: (0, 0)),
            pl.BlockSpec((D, 2 * D), lambda i: (0, 0)),
            pl.BlockSpec((1, 2 * D), lambda i: (0, 0)),
            pl.BlockSpec((D, D), lambda i: (0, 0)),
            pl.BlockSpec((2 * D, D), lambda i: (0, 0)),
            pl.BlockSpec((1, D), lambda i: (0, 0)),
        ],
        out_specs=pl.BlockSpec((BN, D), lambda i: (i, 0)),
        out_shape=jax.ShapeDtypeStruct((N, D), jnp.float32),
    )(x, partials, lw, rb.reshape(1, D), w1xt, w1mt, b1.reshape(1, 2 * D),
      w2xt, w2mt, b2.reshape(1, D))


def kernel(x, edge_index, edges_type, is_block, rel_weight, loop_weight,
           rel_bias, W1, b1, W2, b2):
    del is_block
    src = edge_index[0].astype(jnp.int32)
    dst = edge_index[1].astype(jnp.int32)
    et = edges_type.astype(jnp.int32)
    xw = _relmm(x, rel_weight)
    partials = _sc_agg(xw, src, dst, et)
    w1xt = W1[:, :D].T
    w1mt = W1[:, D:].T
    w2xt = W2[:, :D].T
    w2mt = W2[:, D:].T
    return _mlp(x, partials, loop_weight, rel_bias, w1xt, w1mt, b1,
                w2xt, w2mt, b2)


# speedup vs baseline: 18.1694x; 18.1694x over previous
import functools

import jax
import jax.numpy as jnp
from jax import lax
from jax.experimental import pallas as pl
from jax.experimental.pallas import tpu as pltpu
from jax.experimental.pallas import tpu_sc as plsc

N = 10000
E = 320000
D = 128
R = 4

NC = 2
NS = 16
L = 16
NW = NC * NS
CH = 128
NCHUNK = E // CH
NJ = -(-NCHUNK // NW)
BLK = 80
NBLK = N // BLK
NT = -(-NBLK // NS)


def _relmm(x, rel_weight):
    BN = 1000

    def body(x_ref, w_ref, o_ref):
        o_ref[...] = jnp.dot(x_ref[...], w_ref[0],
                             preferred_element_type=jnp.float32)

    return pl.pallas_call(
        body,
        grid=(R, N // BN),
        in_specs=[
            pl.BlockSpec((BN, D), lambda r, i: (i, 0)),
            pl.BlockSpec((1, D, D), lambda r, i: (r, 0, 0)),
        ],
        out_specs=pl.BlockSpec((BN, D), lambda r, i: (r * (N // BN) + i, 0)),
        out_shape=jax.ShapeDtypeStruct((R * N, D), jnp.float32),
    )(x, rel_weight)


def _sc_agg(xw, src, dst, et):
    mesh = plsc.VectorSubcoreMesh(core_axis_name="c", subcore_axis_name="s")

    @functools.partial(
        pl.kernel,
        mesh=mesh,
        out_type=jax.ShapeDtypeStruct((NC, N, D), jnp.float32),
        scratch_types=[
            pltpu.VMEM((CH,), jnp.int32),
            pltpu.VMEM((CH,), jnp.int32),
            pltpu.VMEM((CH,), jnp.int32),
            pltpu.VMEM((CH,), jnp.int32),
            pltpu.VMEM((CH, D), jnp.float32),
            pltpu.VMEM((BLK, D), jnp.float32),
            pltpu.VMEM_SHARED((N, D), jnp.float32),
            pltpu.SemaphoreType.DMA,
        ],
    )
    def k(xw_hbm, src_hbm, dst_hbm, et_hbm, out_hbm,
          src_v, dst_v, et_v, gidx_v, rows_v, stage_v, agg_sh, sem):
        c = lax.axis_index("c")
        s = lax.axis_index("s")
        wid = s * NC + c

        z16 = jnp.zeros((L,), jnp.float32)

        def zrow(i, carry):
            for j in range(D // L):
                stage_v[i, pl.ds(j * L, L)] = z16
            return carry

        lax.fori_loop(0, BLK, zrow, 0)
        for t in range(NT):
            b = t * NS + s

            @pl.when(b < NBLK)
            def _():
                pltpu.sync_copy(stage_v, agg_sh.at[pl.ds(b * BLK, BLK)])

        plsc.subcore_barrier()

        def chunk(j, carry):
            cid = wid + j * NW

            @pl.when(cid < NCHUNK)
            def _():
                base = cid * CH
                pltpu.sync_copy(src_hbm.at[pl.ds(base, CH)], src_v)
                pltpu.sync_copy(et_hbm.at[pl.ds(base, CH)], et_v)
                pltpu.sync_copy(dst_hbm.at[pl.ds(base, CH)], dst_v)
                for i in range(CH // L):
                    sl = pl.ds(i * L, L)
                    gidx_v[sl] = et_v[sl] * N + src_v[sl]
                pltpu.async_copy(xw_hbm.at[gidx_v], rows_v, sem).wait()
                pltpu.sync_copy(rows_v, agg_sh.at[dst_v], add=True)

            return carry

        lax.fori_loop(0, NJ, chunk, 0)
        plsc.subcore_barrier()

        for t in range(NT):
            b = t * NS + s

            @pl.when(b < NBLK)
            def _():
                r0 = b * BLK
                pltpu.sync_copy(agg_sh.at[pl.ds(r0, BLK)], stage_v)
                pltpu.sync_copy(stage_v, out_hbm.at[c, pl.ds(r0, BLK)])

    return k(xw, src, dst, et)


def _mlp(x, partials, lw, rb, w1xt, w1mt, b1, w2xt, w2mt, b2):
    BN = 1000

    def body(x_ref, p_ref, lw_ref, rb_ref, w1x_ref, w1m_ref, b1_ref,
             w2x_ref, w2m_ref, b2_ref, o_ref):
        xb = x_ref[...]
        msg = (p_ref[0] + p_ref[1]
               + jnp.dot(xb, lw_ref[...], preferred_element_type=jnp.float32)
               + rb_ref[...])
        h = (jnp.dot(xb, w1x_ref[...], preferred_element_type=jnp.float32)
             + jnp.dot(msg, w1m_ref[...], preferred_element_type=jnp.float32)
             + b1_ref[...])
        mid = jnp.tanh(h)
        o_ref[...] = (jnp.dot(xb, w2x_ref[...], preferred_element_type=jnp.float32)
                      + jnp.dot(mid, w2m_ref[...], preferred_element_type=jnp.float32)
                      + b2_ref[...])

    return pl.pallas_call(
        body,
        grid=(N // BN,),
        in_specs=[
            pl.BlockSpec((BN, D), lambda i: (i, 0)),
            pl.BlockSpec((NC, BN, D), lambda i: (0, i, 0)),
            pl.BlockSpec((D, D), lambda i: (0, 0)),
            pl.BlockSpec((1, D), lambda i: (0, 0)),
            pl.BlockSpec((D, 2 * D), lambda i: (0, 0)),
            pl.BlockSpec((D, 2 * D), lambda i: (0, 0)),
            pl.BlockSpec((1, 2 * D), lambda i: (0, 0)),
            pl.BlockSpec((D, D), lambda i: (0, 0)),
            pl.BlockSpec((2 * D, D), lambda i: (0, 0)),
            pl.BlockSpec((1, D), lambda i: (0, 0)),
        ],
        out_specs=pl.BlockSpec((BN, D), lambda i: (i, 0)),
        out_shape=jax.ShapeDtypeStruct((N, D), jnp.float32),
    )(x, partials, lw, rb.reshape(1, D), w1xt, w1mt, b1.reshape(1, 2 * D),
      w2xt, w2mt, b2.reshape(1, D))


def kernel(x, edge_index, edges_type, is_block, rel_weight, loop_weight,
           rel_bias, W1, b1, W2, b2):
    del is_block
    src = edge_index[0].astype(jnp.int32)
    dst = edge_index[1].astype(jnp.int32)
    et = edges_type.astype(jnp.int32)
    xw = _relmm(x, rel_weight)
    partials = _sc_agg(xw, src, dst, et)
    w1xt = W1[:, :D].T
    w1mt = W1[:, D:].T
    w2xt = W2[:, :D].T
    w2mt = W2[:, D:].T
    return _mlp(x, partials, loop_weight, rel_bias, w1xt, w1mt, b1,
                w2xt, w2mt, b2)


# speedup vs baseline: 29.8101x; 1.6407x over previous
import functools

import jax
import jax.numpy as jnp
from jax import lax
from jax.experimental import pallas as pl
from jax.experimental.pallas import tpu as pltpu
from jax.experimental.pallas import tpu_sc as plsc

N = 10000
E = 320000
D = 128
R = 4

NC = 2
NS = 16
L = 16
NW = NC * NS
CH = 128
K = 1
CHOUT = K * CH
NCHUNK = E // CHOUT
NJ = -(-NCHUNK // NW)
NJP = 2 * (-(-NJ // 2))
IW = 3 * CHOUT
BLK = 80
NBLK = N // BLK
NT = -(-NBLK // NS)


def _relmm(x, rel_weight):
    BN = 1000

    def body(x_ref, w_ref, o_ref):
        o_ref[...] = jnp.dot(x_ref[...], w_ref[0],
                             preferred_element_type=jnp.float32)

    return pl.pallas_call(
        body,
        grid=(R, N // BN),
        in_specs=[
            pl.BlockSpec((BN, D), lambda r, i: (i, 0)),
            pl.BlockSpec((1, D, D), lambda r, i: (r, 0, 0)),
        ],
        out_specs=pl.BlockSpec((BN, D), lambda r, i: (r * (N // BN) + i, 0)),
        out_shape=jax.ShapeDtypeStruct((R * N, D), jnp.float32),
    )(x, rel_weight)


def _sc_agg(xw, eidx):
    mesh = plsc.VectorSubcoreMesh(core_axis_name="c", subcore_axis_name="s")

    @functools.partial(
        pl.kernel,
        mesh=mesh,
        out_type=jax.ShapeDtypeStruct((NC, N, D), jnp.float32),
        scratch_types=[
            pltpu.VMEM((IW,), jnp.int32),
            pltpu.VMEM((IW,), jnp.int32),
            pltpu.VMEM((K, CH), jnp.int32),
            pltpu.VMEM((K, CH), jnp.int32),
            pltpu.VMEM((K, CH), jnp.int32),
            pltpu.VMEM((K, CH), jnp.int32),
            pltpu.VMEM((CHOUT, D), jnp.float32),
            pltpu.VMEM((CHOUT, D), jnp.float32),
            pltpu.VMEM_SHARED((N, D), jnp.float32),
            pltpu.SemaphoreType.DMA,
            pltpu.SemaphoreType.DMA,
        ],
    )
    def k(xw_hbm, eidx_hbm, out_hbm,
          idxA, idxB, gixA, gixB, dstA, dstB, rowA, rowB,
          agg_sh, isem, gsem):
        c = lax.axis_index("c")
        s = lax.axis_index("s")
        wid = s * NC + c
        stage_v = rowA.at[pl.ds(0, BLK)]

        z16 = jnp.zeros((L,), jnp.float32)

        def zrow(i, carry):
            for j in range(D // L):
                rowA[i, pl.ds(j * L, L)] = z16
            return carry

        lax.fori_loop(0, BLK, zrow, 0)
        for t in range(NT):
            b = t * NS + s

            @pl.when(b < NBLK)
            def _():
                pltpu.sync_copy(stage_v, agg_sh.at[pl.ds(b * BLK, BLK)])

        plsc.subcore_barrier()

        def build(idx_v, gix_v, dst_v):
            for i in range(CHOUT // L):
                t, p = i // (CH // L), i % (CH // L)
                sl = pl.ds(p * L, L)
                gix_v[t, sl] = (idx_v[pl.ds(CHOUT + i * L, L)] * N
                                + idx_v[pl.ds(i * L, L)])
                dst_v[t, sl] = idx_v[pl.ds(2 * CHOUT + i * L, L)]

        def fetch_idx(cid, idx_v):
            return pltpu.async_copy(
                eidx_hbm.at[pl.ds(cid * IW, IW)], idx_v, isem)

        def gather(gix_v, row_v):
            return [pltpu.async_copy(xw_hbm.at[gix_v.at[t]],
                                     row_v.at[pl.ds(t * CH, CH)], gsem)
                    for t in range(K)]

        def scatter(dst_v, row_v):
            for t in range(K):
                pltpu.sync_copy(row_v.at[pl.ds(t * CH, CH)],
                                agg_sh.at[dst_v.at[t]], add=True)

        fetch_idx(wid, idxA).wait()
        build(idxA, gixA, dstA)
        for h in gather(gixA, rowA):
            h.wait()
        fetch_idx(wid + NW, idxB)

        def half(j, cur):
            idx_c, gix_c, dst_c, row_c = (
                (idxA, gixA, dstA, rowA) if cur == 0
                else (idxB, gixB, dstB, rowB))
            idx_n, gix_n, dst_n, row_n = (
                (idxB, gixB, dstB, rowB) if cur == 0
                else (idxA, gixA, dstA, rowA))
            cid1 = wid + (j + 1) * NW
            cid2 = wid + (j + 2) * NW

            @pl.when(cid1 < NCHUNK)
            def _():
                pltpu.make_async_copy(
                    eidx_hbm.at[pl.ds(0, IW)], idx_n, isem).wait()
                build(idx_n, gix_n, dst_n)
                gather(gix_n, row_n)

            @pl.when(cid2 < NCHUNK)
            def _():
                fetch_idx(cid2, idx_c)

            @pl.when(wid + j * NW < NCHUNK)
            def _():
                scatter(dst_c, row_c)

            @pl.when(cid1 < NCHUNK)
            def _():
                for t in range(K):
                    pltpu.make_async_copy(
                        xw_hbm.at[gix_n.at[t]],
                        row_n.at[pl.ds(t * CH, CH)], gsem).wait()

        def body(jj, carry):
            half(2 * jj, 0)
            half(2 * jj + 1, 1)
            return carry

        lax.fori_loop(0, NJP // 2, body, 0)
        plsc.subcore_barrier()

        for t in range(NT):
            b = t * NS + s

            @pl.when(b < NBLK)
            def _():
                r0 = b * BLK
                pltpu.sync_copy(agg_sh.at[pl.ds(r0, BLK)], stage_v)
                pltpu.sync_copy(stage_v, out_hbm.at[c, pl.ds(r0, BLK)])

    return k(xw, eidx)


def _mlp(x, partials, lw, rb, w1xt, w1mt, b1, w2xt, w2mt, b2):
    BN = 1000

    def body(x_ref, p_ref, lw_ref, rb_ref, w1x_ref, w1m_ref, b1_ref,
             w2x_ref, w2m_ref, b2_ref, o_ref):
        xb = x_ref[...]
        msg = (p_ref[0] + p_ref[1]
               + jnp.dot(xb, lw_ref[...], preferred_element_type=jnp.float32)
               + rb_ref[...])
        h = (jnp.dot(xb, w1x_ref[...], preferred_element_type=jnp.float32)
             + jnp.dot(msg, w1m_ref[...], preferred_element_type=jnp.float32)
             + b1_ref[...])
        mid = jnp.tanh(h)
        o_ref[...] = (jnp.dot(xb, w2x_ref[...], preferred_element_type=jnp.float32)
                      + jnp.dot(mid, w2m_ref[...], preferred_element_type=jnp.float32)
                      + b2_ref[...])

    return pl.pallas_call(
        body,
        grid=(N // BN,),
        in_specs=[
            pl.BlockSpec((BN, D), lambda i: (i, 0)),
            pl.BlockSpec((NC, BN, D), lambda i: (0, i, 0)),
            pl.BlockSpec((D, D), lambda i: (0, 0)),
            pl.BlockSpec((1, D), lambda i: (0, 0)),
            pl.BlockSpec((D, 2 * D), lambda i: (0, 0)),
            pl.BlockSpec((D, 2 * D), lambda i: (0, 0)),
            pl.BlockSpec((1, 2 * D), lambda i: (0, 0)),
            pl.BlockSpec((D, D), lambda i: (0, 0)),
            pl.BlockSpec((2 * D, D), lambda i: (0, 0)),
            pl.BlockSpec((1, D), lambda i: (0, 0)),
        ],
        out_specs=pl.BlockSpec((BN, D), lambda i: (i, 0)),
        out_shape=jax.ShapeDtypeStruct((N, D), jnp.float32),
    )(x, partials, lw, rb.reshape(1, D), w1xt, w1mt, b1.reshape(1, 2 * D),
      w2xt, w2mt, b2.reshape(1, D))


def kernel(x, edge_index, edges_type, is_block, rel_weight, loop_weight,
           rel_bias, W1, b1, W2, b2):
    del is_block
    src = edge_index[0].astype(jnp.int32)
    dst = edge_index[1].astype(jnp.int32)
    et = edges_type.astype(jnp.int32)
    eidx = (jnp.stack([src, et, dst])
            .reshape(3, NCHUNK, CHOUT)
            .transpose(1, 0, 2)
            .reshape(-1))
    xw = _relmm(x, rel_weight)
    partials = _sc_agg(xw, eidx)
    w1xt = W1[:, :D].T
    w1mt = W1[:, D:].T
    w2xt = W2[:, :D].T
    w2mt = W2[:, D:].T
    return _mlp(x, partials, loop_weight, rel_bias, w1xt, w1mt, b1,
                w2xt, w2mt, b2)


# speedup vs baseline: 29.8558x; 1.0015x over previous
import functools

import jax
import jax.numpy as jnp
from jax import lax
from jax.experimental import pallas as pl
from jax.experimental.pallas import tpu as pltpu
from jax.experimental.pallas import tpu_sc as plsc

N = 10000
E = 320000
D = 128
R = 4

NC = 2
NS = 16
L = 16
NW = NC * NS
CH = 128
K = 1
CHOUT = K * CH
NCHUNK = E // CHOUT
NJ = -(-NCHUNK // NW)
NJP = 2 * (-(-NJ // 2))
IW = 3 * CHOUT
BLK = 80
NBLK = N // BLK
NT = -(-NBLK // NS)


def _relmm(x, rel_weight):
    BN = 1000

    def body(x_ref, w_ref, o_ref):
        o_ref[...] = jnp.dot(x_ref[...], w_ref[0],
                             preferred_element_type=jnp.float32)

    return pl.pallas_call(
        body,
        grid=(R, N // BN),
        in_specs=[
            pl.BlockSpec((BN, D), lambda r, i: (i, 0)),
            pl.BlockSpec((1, D, D), lambda r, i: (r, 0, 0)),
        ],
        out_specs=pl.BlockSpec((BN, D), lambda r, i: (r * (N // BN) + i, 0)),
        out_shape=jax.ShapeDtypeStruct((R * N, D), jnp.float32),
    )(x, rel_weight)


def _sc_agg(xw, eidx):
    mesh = plsc.VectorSubcoreMesh(core_axis_name="c", subcore_axis_name="s")

    @functools.partial(
        pl.kernel,
        mesh=mesh,
        out_type=jax.ShapeDtypeStruct((NC, N, D), jnp.float32),
        scratch_types=[
            pltpu.VMEM((IW,), jnp.int32),
            pltpu.VMEM((IW,), jnp.int32),
            pltpu.VMEM((K, CH), jnp.int32),
            pltpu.VMEM((K, CH), jnp.int32),
            pltpu.VMEM((K, CH), jnp.int32),
            pltpu.VMEM((K, CH), jnp.int32),
            pltpu.VMEM((CHOUT, D), jnp.float32),
            pltpu.VMEM((CHOUT, D), jnp.float32),
            pltpu.VMEM_SHARED((N, D), jnp.float32),
            pltpu.SemaphoreType.DMA,
            pltpu.SemaphoreType.DMA,
            pltpu.SemaphoreType.DMA,
        ],
    )
    def k(xw_hbm, eidx_hbm, out_hbm,
          idxA, idxB, gixA, gixB, dstA, dstB, rowA, rowB,
          agg_sh, isem, gsem, ssem):
        c = lax.axis_index("c")
        s = lax.axis_index("s")
        wid = s * NC + c
        stage_v = rowA.at[pl.ds(0, BLK)]

        z16 = jnp.zeros((L,), jnp.float32)

        def zrow(i, carry):
            for j in range(D // L):
                rowA[i, pl.ds(j * L, L)] = z16
            return carry

        lax.fori_loop(0, BLK, zrow, 0)
        for t in range(NT):
            b = t * NS + s

            @pl.when(b < NBLK)
            def _():
                pltpu.sync_copy(stage_v, agg_sh.at[pl.ds(b * BLK, BLK)])

        plsc.subcore_barrier()

        def build(idx_v, gix_v, dst_v):
            for i in range(CHOUT // L):
                t, p = i // (CH // L), i % (CH // L)
                sl = pl.ds(p * L, L)
                gix_v[t, sl] = (idx_v[pl.ds(CHOUT + i * L, L)] * N
                                + idx_v[pl.ds(i * L, L)])
                dst_v[t, sl] = idx_v[pl.ds(2 * CHOUT + i * L, L)]

        def fetch_idx(cid, idx_v):
            return pltpu.async_copy(
                eidx_hbm.at[pl.ds(cid * IW, IW)], idx_v, isem)

        def gather(gix_v, row_v):
            for t in range(K):
                pltpu.async_copy(xw_hbm.at[gix_v.at[t]],
                                 row_v.at[pl.ds(t * CH, CH)], gsem)

        fetch_idx(wid, idxA).wait()
        build(idxA, gixA, dstA)
        gather(gixA, rowA)
        fetch_idx(wid + NW, idxB)

        def half(j, cur):
            idx_c, gix_c, dst_c, row_c = (
                (idxA, gixA, dstA, rowA) if cur == 0
                else (idxB, gixB, dstB, rowB))
            idx_n, gix_n, dst_n, row_n = (
                (idxB, gixB, dstB, rowB) if cur == 0
                else (idxA, gixA, dstA, rowA))
            cid0 = wid + j * NW
            cid1 = wid + (j + 1) * NW
            cid2 = wid + (j + 2) * NW

            @pl.when((j >= 1) & (wid + (j - 1) * NW < NCHUNK))
            def _():
                for t in range(K):
                    pltpu.make_async_copy(
                        row_n.at[pl.ds(t * CH, CH)],
                        agg_sh.at[dst_n.at[t]], ssem).wait()

            @pl.when(cid1 < NCHUNK)
            def _():
                pltpu.make_async_copy(
                    eidx_hbm.at[pl.ds(0, IW)], idx_n, isem).wait()
                build(idx_n, gix_n, dst_n)

            @pl.when(cid0 < NCHUNK)
            def _():
                for t in range(K):
                    pltpu.make_async_copy(
                        xw_hbm.at[gix_c.at[t]],
                        row_c.at[pl.ds(t * CH, CH)], gsem).wait()
                for t in range(K):
                    pltpu.async_copy(row_c.at[pl.ds(t * CH, CH)],
                                     agg_sh.at[dst_c.at[t]], ssem, add=True)

            @pl.when(cid1 < NCHUNK)
            def _():
                gather(gix_n, row_n)

            @pl.when(cid2 < NCHUNK)
            def _():
                fetch_idx(cid2, idx_c)

        def body(jj, carry):
            half(2 * jj, 0)
            half(2 * jj + 1, 1)
            return carry

        lax.fori_loop(0, NJP // 2, body, 0)
        plsc.subcore_barrier()

        for t in range(NT):
            b = t * NS + s

            @pl.when(b < NBLK)
            def _():
                r0 = b * BLK
                pltpu.sync_copy(agg_sh.at[pl.ds(r0, BLK)], stage_v)
                pltpu.sync_copy(stage_v, out_hbm.at[c, pl.ds(r0, BLK)])

    return k(xw, eidx)


def _mlp(x, partials, lw, rb, w1xt, w1mt, b1, w2xt, w2mt, b2):
    BN = 1000

    def body(x_ref, p_ref, lw_ref, rb_ref, w1x_ref, w1m_ref, b1_ref,
             w2x_ref, w2m_ref, b2_ref, o_ref):
        xb = x_ref[...]
        msg = (p_ref[0] + p_ref[1]
               + jnp.dot(xb, lw_ref[...], preferred_element_type=jnp.float32)
               + rb_ref[...])
        h = (jnp.dot(xb, w1x_ref[...], preferred_element_type=jnp.float32)
             + jnp.dot(msg, w1m_ref[...], preferred_element_type=jnp.float32)
             + b1_ref[...])
        mid = jnp.tanh(h)
        o_ref[...] = (jnp.dot(xb, w2x_ref[...], preferred_element_type=jnp.float32)
                      + jnp.dot(mid, w2m_ref[...], preferred_element_type=jnp.float32)
                      + b2_ref[...])

    return pl.pallas_call(
        body,
        grid=(N // BN,),
        in_specs=[
            pl.BlockSpec((BN, D), lambda i: (i, 0)),
            pl.BlockSpec((NC, BN, D), lambda i: (0, i, 0)),
            pl.BlockSpec((D, D), lambda i: (0, 0)),
            pl.BlockSpec((1, D), lambda i: (0, 0)),
            pl.BlockSpec((D, 2 * D), lambda i: (0, 0)),
            pl.BlockSpec((D, 2 * D), lambda i: (0, 0)),
            pl.BlockSpec((1, 2 * D), lambda i: (0, 0)),
            pl.BlockSpec((D, D), lambda i: (0, 0)),
            pl.BlockSpec((2 * D, D), lambda i: (0, 0)),
            pl.BlockSpec((1, D), lambda i: (0, 0)),
        ],
        out_specs=pl.BlockSpec((BN, D), lambda i: (i, 0)),
        out_shape=jax.ShapeDtypeStruct((N, D), jnp.float32),
    )(x, partials, lw, rb.reshape(1, D), w1xt, w1mt, b1.reshape(1, 2 * D),
      w2xt, w2mt, b2.reshape(1, D))


def kernel(x, edge_index, edges_type, is_block, rel_weight, loop_weight,
           rel_bias, W1, b1, W2, b2):
    del is_block
    src = edge_index[0].astype(jnp.int32)
    dst = edge_index[1].astype(jnp.int32)
    et = edges_type.astype(jnp.int32)
    eidx = (jnp.stack([src, et, dst])
            .reshape(3, NCHUNK, CHOUT)
            .transpose(1, 0, 2)
            .reshape(-1))
    xw = _relmm(x, rel_weight)
    partials = _sc_agg(xw, eidx)
    w1xt = W1[:, :D].T
    w1mt = W1[:, D:].T
    w2xt = W2[:, :D].T
    w2mt = W2[:, D:].T
    return _mlp(x, partials, loop_weight, rel_bias, w1xt, w1mt, b1,
                w2xt, w2mt, b2)


# speedup vs baseline: 35.1104x; 1.1760x over previous
import functools

import jax
import jax.numpy as jnp
from jax import lax
from jax.experimental import pallas as pl
from jax.experimental.pallas import tpu as pltpu
from jax.experimental.pallas import tpu_sc as plsc

N = 10000
E = 320000
D = 128
R = 4

NC = 2
NS = 16
L = 16
NW = NC * NS
CH = 128
K = 1
CHOUT = K * CH
NCHUNK = E // CHOUT
NJ = -(-NCHUNK // NW)
NJP = 2 * (-(-NJ // 2))
IW = 3 * CHOUT
BLK = 80
NBLK = N // BLK
NT = -(-NBLK // NS)


def _relmm(x, rel_weight):
    BN = 1000

    def body(x_ref, w_ref, o_ref):
        xb = x_ref[...]
        for r in range(R):
            o_ref[r] = jnp.dot(xb, w_ref[r],
                               preferred_element_type=jnp.float32)

    return pl.pallas_call(
        body,
        grid=(N // BN,),
        in_specs=[
            pl.BlockSpec((BN, D), lambda i: (i, 0)),
            pl.BlockSpec((R, D, D), lambda i: (0, 0, 0)),
        ],
        out_specs=pl.BlockSpec((R, BN, D), lambda i: (0, i, 0)),
        out_shape=jax.ShapeDtypeStruct((R, N, D), jnp.float32),
    )(x, rel_weight)


def _sc_agg(xw, src, et, dst):
    mesh = plsc.VectorSubcoreMesh(core_axis_name="c", subcore_axis_name="s")

    @functools.partial(
        pl.kernel,
        mesh=mesh,
        out_type=jax.ShapeDtypeStruct((NC, N, D), jnp.float32),
        scratch_types=[
            pltpu.VMEM((CH,), jnp.int32),
            pltpu.VMEM((CH,), jnp.int32),
            pltpu.VMEM((CH,), jnp.int32),
            pltpu.VMEM((CH,), jnp.int32),
            pltpu.VMEM((CH,), jnp.int32),
            pltpu.VMEM((CH,), jnp.int32),
            pltpu.VMEM((CH,), jnp.int32),
            pltpu.VMEM((CH,), jnp.int32),
            pltpu.VMEM((CH,), jnp.int32),
            pltpu.VMEM((CH,), jnp.int32),
            pltpu.VMEM((CHOUT, D), jnp.float32),
            pltpu.VMEM((CHOUT, D), jnp.float32),
            pltpu.VMEM_SHARED((N, D), jnp.float32),
            pltpu.SemaphoreType.DMA,
            pltpu.SemaphoreType.DMA,
            pltpu.SemaphoreType.DMA,
        ],
    )
    def k(xw_hbm, src_hbm, et_hbm, dst_hbm, out_hbm,
          srcA, srcB, etA, etB, drwA, drwB, gixA, gixB, dstA, dstB,
          rowA, rowB, agg_sh, isem, gsem, ssem):
        c = lax.axis_index("c")
        s = lax.axis_index("s")
        wid = s * NC + c
        stage_v = rowA.at[pl.ds(0, BLK)]

        z16 = jnp.zeros((L,), jnp.float32)

        def zrow(i, carry):
            for j in range(D // L):
                rowA[i, pl.ds(j * L, L)] = z16
            return carry

        lax.fori_loop(0, BLK, zrow, 0)
        for t in range(NT):
            b = t * NS + s

            @pl.when(b < NBLK)
            def _():
                pltpu.sync_copy(stage_v, agg_sh.at[pl.ds(b * BLK, BLK)])

        plsc.subcore_barrier()

        def build(src_v, et_v, drw_v, gix_v, dst_v):
            for i in range(CH // L):
                sl = pl.ds(i * L, L)
                gix_v[sl] = et_v[sl] * N + src_v[sl]
                dst_v[sl] = drw_v[sl]

        def fetch_idx(cid, src_v, et_v, drw_v):
            base = cid * CH
            pltpu.async_copy(src_hbm.at[pl.ds(base, CH)], src_v, isem)
            pltpu.async_copy(et_hbm.at[pl.ds(base, CH)], et_v, isem)
            pltpu.async_copy(dst_hbm.at[pl.ds(base, CH)], drw_v, isem)

        def drain_idx(src_v, et_v, drw_v):
            pltpu.make_async_copy(src_hbm.at[pl.ds(0, CH)], src_v, isem).wait()
            pltpu.make_async_copy(et_hbm.at[pl.ds(0, CH)], et_v, isem).wait()
            pltpu.make_async_copy(dst_hbm.at[pl.ds(0, CH)], drw_v, isem).wait()

        fetch_idx(wid, srcA, etA, drwA)
        drain_idx(srcA, etA, drwA)
        build(srcA, etA, drwA, gixA, dstA)
        pltpu.async_copy(xw_hbm.at[gixA], rowA, gsem)
        fetch_idx(wid + NW, srcB, etB, drwB)

        def half(j, cur):
            src_c, et_c, drw_c, gix_c, dst_c, row_c = (
                (srcA, etA, drwA, gixA, dstA, rowA) if cur == 0
                else (srcB, etB, drwB, gixB, dstB, rowB))
            src_n, et_n, drw_n, gix_n, dst_n, row_n = (
                (srcB, etB, drwB, gixB, dstB, rowB) if cur == 0
                else (srcA, etA, drwA, gixA, dstA, rowA))
            cid0 = wid + j * NW
            cid1 = wid + (j + 1) * NW
            cid2 = wid + (j + 2) * NW

            @pl.when((j >= 1) & (wid + (j - 1) * NW < NCHUNK))
            def _():
                pltpu.make_async_copy(row_n, agg_sh.at[dst_n], ssem).wait()

            @pl.when(cid1 < NCHUNK)
            def _():
                drain_idx(src_n, et_n, drw_n)
                build(src_n, et_n, drw_n, gix_n, dst_n)

            @pl.when(cid0 < NCHUNK)
            def _():
                pltpu.make_async_copy(xw_hbm.at[gix_c], row_c, gsem).wait()
                pltpu.async_copy(row_c, agg_sh.at[dst_c], ssem, add=True)

            @pl.when(cid1 < NCHUNK)
            def _():
                pltpu.async_copy(xw_hbm.at[gix_n], row_n, gsem)

            @pl.when(cid2 < NCHUNK)
            def _():
                fetch_idx(cid2, src_c, et_c, drw_c)

        def body(jj, carry):
            half(2 * jj, 0)
            half(2 * jj + 1, 1)
            return carry

        lax.fori_loop(0, NJP // 2, body, 0)
        plsc.subcore_barrier()

        for t in range(NT):
            b = t * NS + s

            @pl.when(b < NBLK)
            def _():
                r0 = b * BLK
                pltpu.sync_copy(agg_sh.at[pl.ds(r0, BLK)], stage_v)
                pltpu.sync_copy(stage_v, out_hbm.at[c, pl.ds(r0, BLK)])

    return k(xw, src, et, dst)


def _mlp(x, partials, lw, rb, w1xt, w1mt, b1, w2xt, w2mt, b2):
    BN = 1000

    def body(x_ref, p_ref, lw_ref, rb_ref, w1x_ref, w1m_ref, b1_ref,
             w2x_ref, w2m_ref, b2_ref, o_ref):
        xb = x_ref[...]
        msg = (p_ref[0] + p_ref[1]
               + jnp.dot(xb, lw_ref[...], preferred_element_type=jnp.float32)
               + rb_ref[...])
        h = (jnp.dot(xb, w1x_ref[...], preferred_element_type=jnp.float32)
             + jnp.dot(msg, w1m_ref[...], preferred_element_type=jnp.float32)
             + b1_ref[...])
        mid = jnp.tanh(h)
        o_ref[...] = (jnp.dot(xb, w2x_ref[...], preferred_element_type=jnp.float32)
                      + jnp.dot(mid, w2m_ref[...], preferred_element_type=jnp.float32)
                      + b2_ref[...])

    return pl.pallas_call(
        body,
        grid=(N // BN,),
        in_specs=[
            pl.BlockSpec((BN, D), lambda i: (i, 0)),
            pl.BlockSpec((NC, BN, D), lambda i: (0, i, 0)),
            pl.BlockSpec((D, D), lambda i: (0, 0)),
            pl.BlockSpec((1, D), lambda i: (0, 0)),
            pl.BlockSpec((D, 2 * D), lambda i: (0, 0)),
            pl.BlockSpec((D, 2 * D), lambda i: (0, 0)),
            pl.BlockSpec((1, 2 * D), lambda i: (0, 0)),
            pl.BlockSpec((D, D), lambda i: (0, 0)),
            pl.BlockSpec((2 * D, D), lambda i: (0, 0)),
            pl.BlockSpec((1, D), lambda i: (0, 0)),
        ],
        out_specs=pl.BlockSpec((BN, D), lambda i: (i, 0)),
        out_shape=jax.ShapeDtypeStruct((N, D), jnp.float32),
    )(x, partials, lw, rb.reshape(1, D), w1xt, w1mt, b1.reshape(1, 2 * D),
      w2xt, w2mt, b2.reshape(1, D))


def kernel(x, edge_index, edges_type, is_block, rel_weight, loop_weight,
           rel_bias, W1, b1, W2, b2):
    del is_block
    src = edge_index[0].astype(jnp.int32)
    dst = edge_index[1].astype(jnp.int32)
    et = edges_type.astype(jnp.int32)
    xw = _relmm(x, rel_weight).reshape(R * N, D)
    partials = _sc_agg(xw, src, et, dst)
    w1xt = W1[:, :D].T
    w1mt = W1[:, D:].T
    w2xt = W2[:, :D].T
    w2mt = W2[:, D:].T
    return _mlp(x, partials, loop_weight, rel_bias, w1xt, w1mt, b1,
                w2xt, w2mt, b2)


# speedup vs baseline: 37.1571x; 1.0583x over previous
import functools

import jax
import jax.numpy as jnp
from jax import lax
from jax.experimental import pallas as pl
from jax.experimental.pallas import tpu as pltpu
from jax.experimental.pallas import tpu_sc as plsc

N = 10000
E = 320000
D = 128
R = 4

NC = 2
NS = 16
L = 16
NW = NC * NS
CH = 128
K = 1
CHOUT = K * CH
NCHUNK = E // CHOUT
NJ = -(-NCHUNK // NW)
NJP = 2 * (-(-NJ // 2))
IW = 3 * CHOUT
BLK = 80
NBLK = N // BLK
NT = -(-NBLK // NS)


def _relmm(x, rel_weight):
    BN = 1000

    def body(x_ref, w_ref, o_ref):
        xb = x_ref[...]
        for r in range(R):
            o_ref[r] = jnp.dot(xb, w_ref[r],
                               preferred_element_type=jnp.float32)

    return pl.pallas_call(
        body,
        grid=(N // BN,),
        in_specs=[
            pl.BlockSpec((BN, D), lambda i: (i, 0)),
            pl.BlockSpec((R, D, D), lambda i: (0, 0, 0)),
        ],
        out_specs=pl.BlockSpec((R, BN, D), lambda i: (0, i, 0)),
        out_shape=jax.ShapeDtypeStruct((R, N, D), jnp.float32),
    )(x, rel_weight)


def _sc_agg(xw, ei, et):
    mesh = plsc.VectorSubcoreMesh(core_axis_name="c", subcore_axis_name="s")

    @functools.partial(
        pl.kernel,
        mesh=mesh,
        out_type=jax.ShapeDtypeStruct((NC, N, D), jnp.float32),
        scratch_types=[
            pltpu.VMEM((CH,), jnp.int32),
            pltpu.VMEM((CH,), jnp.int32),
            pltpu.VMEM((CH,), jnp.int32),
            pltpu.VMEM((CH,), jnp.int32),
            pltpu.VMEM((CH,), jnp.int32),
            pltpu.VMEM((CH,), jnp.int32),
            pltpu.VMEM((CH,), jnp.int32),
            pltpu.VMEM((CH,), jnp.int32),
            pltpu.VMEM((CH,), jnp.int32),
            pltpu.VMEM((CH,), jnp.int32),
            pltpu.VMEM((CHOUT, D), jnp.float32),
            pltpu.VMEM((CHOUT, D), jnp.float32),
            pltpu.VMEM_SHARED((N, D), jnp.float32),
            pltpu.SemaphoreType.DMA,
            pltpu.SemaphoreType.DMA,
            pltpu.SemaphoreType.DMA,
        ],
    )
    def k(xw_hbm, ei_hbm, et_hbm, out_hbm,
          srcA, srcB, etA, etB, drwA, drwB, gixA, gixB, dstA, dstB,
          rowA, rowB, agg_sh, isem, gsem, ssem):
        c = lax.axis_index("c")
        s = lax.axis_index("s")
        wid = s * NC + c
        stage_v = rowA.at[pl.ds(0, BLK)]

        z16 = jnp.zeros((L,), jnp.float32)

        def zrow(i, carry):
            for j in range(D // L):
                rowA[i, pl.ds(j * L, L)] = z16
            return carry

        lax.fori_loop(0, BLK, zrow, 0)
        for t in range(NT):
            b = t * NS + s

            @pl.when(b < NBLK)
            def _():
                pltpu.sync_copy(stage_v, agg_sh.at[pl.ds(b * BLK, BLK)])

        plsc.subcore_barrier()

        def build(src_v, et_v, drw_v, gix_v, dst_v):
            for i in range(CH // L):
                sl = pl.ds(i * L, L)
                gix_v[sl] = et_v[sl] * N + src_v[sl]
                dst_v[sl] = drw_v[sl]

        def fetch_idx(cid, src_v, et_v, drw_v):
            base = cid * CH
            pltpu.async_copy(ei_hbm.at[pl.ds(base, CH)], src_v, isem)
            pltpu.async_copy(et_hbm.at[pl.ds(base, CH)], et_v, isem)
            pltpu.async_copy(ei_hbm.at[pl.ds(E + base, CH)], drw_v, isem)

        def drain_idx(src_v, et_v, drw_v):
            pltpu.make_async_copy(ei_hbm.at[pl.ds(0, CH)], src_v, isem).wait()
            pltpu.make_async_copy(et_hbm.at[pl.ds(0, CH)], et_v, isem).wait()
            pltpu.make_async_copy(ei_hbm.at[pl.ds(0, CH)], drw_v, isem).wait()

        fetch_idx(wid, srcA, etA, drwA)
        drain_idx(srcA, etA, drwA)
        build(srcA, etA, drwA, gixA, dstA)
        pltpu.async_copy(xw_hbm.at[gixA], rowA, gsem)
        fetch_idx(wid + NW, srcB, etB, drwB)

        def half(j, cur):
            src_c, et_c, drw_c, gix_c, dst_c, row_c = (
                (srcA, etA, drwA, gixA, dstA, rowA) if cur == 0
                else (srcB, etB, drwB, gixB, dstB, rowB))
            src_n, et_n, drw_n, gix_n, dst_n, row_n = (
                (srcB, etB, drwB, gixB, dstB, rowB) if cur == 0
                else (srcA, etA, drwA, gixA, dstA, rowA))
            cid0 = wid + j * NW
            cid1 = wid + (j + 1) * NW
            cid2 = wid + (j + 2) * NW

            @pl.when((j >= 1) & (wid + (j - 1) * NW < NCHUNK))
            def _():
                pltpu.make_async_copy(row_n, agg_sh.at[dst_n], ssem).wait()

            @pl.when(cid1 < NCHUNK)
            def _():
                drain_idx(src_n, et_n, drw_n)
                build(src_n, et_n, drw_n, gix_n, dst_n)

            @pl.when(cid0 < NCHUNK)
            def _():
                pltpu.make_async_copy(xw_hbm.at[gix_c], row_c, gsem).wait()
                pltpu.async_copy(row_c, agg_sh.at[dst_c], ssem, add=True)

            @pl.when(cid1 < NCHUNK)
            def _():
                pltpu.async_copy(xw_hbm.at[gix_n], row_n, gsem)

            @pl.when(cid2 < NCHUNK)
            def _():
                fetch_idx(cid2, src_c, et_c, drw_c)

        def body(jj, carry):
            half(2 * jj, 0)
            half(2 * jj + 1, 1)
            return carry

        lax.fori_loop(0, NJP // 2, body, 0)
        plsc.subcore_barrier()

        for t in range(NT):
            b = t * NS + s

            @pl.when(b < NBLK)
            def _():
                r0 = b * BLK
                pltpu.sync_copy(agg_sh.at[pl.ds(r0, BLK)],
                                out_hbm.at[c, pl.ds(r0, BLK)])

    return k(xw, ei, et)


def _mlp(x, partials, lw, rb, w1xt, w1mt, b1, w2xt, w2mt, b2):
    BN = 1000

    def body(x_ref, p_ref, lw_ref, rb_ref, w1x_ref, w1m_ref, b1_ref,
             w2x_ref, w2m_ref, b2_ref, o_ref):
        xb = x_ref[...]
        msg = (p_ref[0] + p_ref[1]
               + jnp.dot(xb, lw_ref[...], preferred_element_type=jnp.float32)
               + rb_ref[...])
        h = (jnp.dot(xb, w1x_ref[...], preferred_element_type=jnp.float32)
             + jnp.dot(msg, w1m_ref[...], preferred_element_type=jnp.float32)
             + b1_ref[...])
        mid = jnp.tanh(h)
        o_ref[...] = (jnp.dot(xb, w2x_ref[...], preferred_element_type=jnp.float32)
                      + jnp.dot(mid, w2m_ref[...], preferred_element_type=jnp.float32)
                      + b2_ref[...])

    return pl.pallas_call(
        body,
        grid=(N // BN,),
        in_specs=[
            pl.BlockSpec((BN, D), lambda i: (i, 0)),
            pl.BlockSpec((NC, BN, D), lambda i: (0, i, 0)),
            pl.BlockSpec((D, D), lambda i: (0, 0)),
            pl.BlockSpec((1, D), lambda i: (0, 0)),
            pl.BlockSpec((D, 2 * D), lambda i: (0, 0)),
            pl.BlockSpec((D, 2 * D), lambda i: (0, 0)),
            pl.BlockSpec((1, 2 * D), lambda i: (0, 0)),
            pl.BlockSpec((D, D), lambda i: (0, 0)),
            pl.BlockSpec((2 * D, D), lambda i: (0, 0)),
            pl.BlockSpec((1, D), lambda i: (0, 0)),
        ],
        out_specs=pl.BlockSpec((BN, D), lambda i: (i, 0)),
        out_shape=jax.ShapeDtypeStruct((N, D), jnp.float32),
    )(x, partials, lw, rb.reshape(1, D), w1xt, w1mt, b1.reshape(1, 2 * D),
      w2xt, w2mt, b2.reshape(1, D))


def kernel(x, edge_index, edges_type, is_block, rel_weight, loop_weight,
           rel_bias, W1, b1, W2, b2):
    del is_block
    ei = edge_index.astype(jnp.int32).reshape(2 * E)
    et = edges_type.astype(jnp.int32)
    xw = _relmm(x, rel_weight).reshape(R * N, D)
    partials = _sc_agg(xw, ei, et)
    w1xt = W1[:, :D].T
    w1mt = W1[:, D:].T
    w2xt = W2[:, :D].T
    w2mt = W2[:, D:].T
    return _mlp(x, partials, loop_weight, rel_bias, w1xt, w1mt, b1,
                w2xt, w2mt, b2)
